# baseline (speedup 1.0000x reference)
import functools

import jax
import jax.numpy as jnp
from jax import lax
from jax.experimental import pallas as pl
from jax.experimental.pallas import tpu as pltpu
from jax.experimental.pallas import tpu_sc as plsc

N = 100000
E = 3200000
D = 16

NC = 2
NS = 16
NW = NC * NS

SUB = 128
SUPER = 8
SUPER_C = 4
CHUNK_E = SUPER_C * SUB
QBLKS = E // (SUPER * SUB)
HB = E // CHUNK_E
HBASE = HB // NW
HEXTRA = HB - HBASE * NW

_SPLIT = [6248] * (NS - 1) + [N - 6248 * (NS - 1)]
_STARTS = [6248 * i for i in range(NS)]


def _sc_aggregate(edge4d, w1d, emb):
  mesh = plsc.VectorSubcoreMesh(core_axis_name="c", subcore_axis_name="s")

  @functools.partial(
      pl.kernel,
      out_type=jax.ShapeDtypeStruct((NC, N, D), jnp.float32),
      mesh=mesh,
      scratch_types=[
          pltpu.VMEM_SHARED((N, D), jnp.float32),
          pltpu.VMEM((SUPER_C, SUB), jnp.int32),
          pltpu.VMEM((SUPER_C, SUB), jnp.int32),
          pltpu.VMEM((CHUNK_E,), jnp.float32),
          pltpu.VMEM((CHUNK_E, D), jnp.float32),
          pltpu.VMEM((SUPER_C, SUB), jnp.int32),
          pltpu.VMEM((SUPER_C, SUB), jnp.int32),
          pltpu.VMEM((CHUNK_E,), jnp.float32),
          pltpu.VMEM((CHUNK_E, D), jnp.float32),
          pltpu.SemaphoreType.DMA,
          pltpu.SemaphoreType.DMA,
          pltpu.SemaphoreType.DMA,
          pltpu.SemaphoreType.DMA,
          pltpu.SemaphoreType.DMA,
          pltpu.SemaphoreType.DMA,
      ],
      compiler_params=pltpu.CompilerParams(use_tc_tiling_on_sc=False),
  )
  def k(edge_hbm, w_hbm, emb_hbm, agg_hbm,
        agg_sh, src_a, dst_a, w_a, rows_a, src_b, dst_b, w_b, rows_b,
        gsem_a, gsem_b, ssem_a, ssem_b, stg_a, stg_b):
    c = lax.axis_index("c")
    s = lax.axis_index("s")
    wid = c * NS + s

    @pl.loop(0, CHUNK_E)
    def _zero_buf(i):
      rows_a[i, :] = jnp.zeros((D,), jnp.float32)

    for ss in range(NS):
      @pl.when(s == ss)
      def _zero_range(start=_STARTS[ss], size=_SPLIT[ss]):
        full, rem = size // CHUNK_E, size % CHUNK_E
        for kk in range(full):
          pltpu.sync_copy(rows_a.at[pl.ds(0, CHUNK_E)],
                          agg_sh.at[pl.ds(start + kk * CHUNK_E, CHUNK_E)])
        if rem:
          pltpu.sync_copy(rows_a.at[pl.ds(0, rem)],
                          agg_sh.at[pl.ds(start + full * CHUNK_E, rem)])
    plsc.subcore_barrier()

    hstart = HBASE * wid + jnp.minimum(wid, HEXTRA)
    hcount = HBASE + jnp.where(wid < HEXTRA, 1, 0)
    npairs = (hcount + 1) // 2

    def stage_issue(h, src_v, dst_v, w_v, sem):
      q = h // 2
      half = (h % 2) * SUPER_C
      pltpu.async_copy(edge_hbm.at[0, q, pl.ds(half, SUPER_C)], src_v, sem)
      pltpu.async_copy(edge_hbm.at[1, q, pl.ds(half, SUPER_C)], dst_v, sem)
      pltpu.async_copy(w_hbm.at[pl.ds(h * CHUNK_E, CHUNK_E)], w_v, sem)

    def stage_wait(src_v, dst_v, w_v, sem):
      pltpu.make_async_copy(edge_hbm.at[0, 0, pl.ds(0, SUPER_C)], src_v, sem).wait()
      pltpu.make_async_copy(edge_hbm.at[1, 0, pl.ds(0, SUPER_C)], dst_v, sem).wait()
      pltpu.make_async_copy(w_hbm.at[pl.ds(0, CHUNK_E)], w_v, sem).wait()

    def fire(src_v, rows_v, sem):
      @pl.loop(0, SUPER_C)
      def _f(j):
        pltpu.async_copy(emb_hbm.at[src_v.at[j]],
                         rows_v.at[pl.ds(j * SUB, SUB)], sem)

    def drain_gathers(src_v, rows_v, sem):
      @pl.loop(0, SUPER_C)
      def _d(j):
        pltpu.make_async_copy(emb_hbm.at[src_v.at[j]],
                              rows_v.at[pl.ds(j * SUB, SUB)], sem).wait()

    def scale(rows_v, w_v):
      @pl.loop(0, CHUNK_E // 16)
      def _t(t):
        base = t * 16
        w16 = w_v[pl.ds(base, 16)]
        for e in range(16):
          rows_v[base + e, :] = rows_v[base + e, :] * jnp.broadcast_to(
              w16[e], (D,))

    def scatter_issue(rows_v, dst_v, sem):
      @pl.loop(0, SUPER_C)
      def _s(j):
        pltpu.async_copy(rows_v.at[pl.ds(j * SUB, SUB)],
                         agg_sh.at[dst_v.at[j]], sem, add=True)

    def scatter_drain(rows_v, dst_v, sem):
      @pl.loop(0, SUPER_C)
      def _s(j):
        pltpu.make_async_copy(rows_v.at[pl.ds(j * SUB, SUB)],
                              agg_sh.at[dst_v.at[j]], sem).wait()

    stage_issue(hstart, src_a, dst_a, w_a, stg_a)
    stage_wait(src_a, dst_a, w_a, stg_a)
    fire(src_a, rows_a, gsem_a)
    stage_issue(hstart + 1, src_b, dst_b, w_b, stg_b)

    @pl.loop(0, npairs)
    def _pair(t):
      h0 = 2 * t
      h1 = h0 + 1
      h2 = h0 + 2
      h3 = h0 + 3

      drain_gathers(src_a, rows_a, gsem_a)

      @pl.when(h1 < hcount)
      def _b_in():
        stage_wait(src_b, dst_b, w_b, stg_b)
        fire(src_b, rows_b, gsem_b)

      scale(rows_a, w_a)
      scatter_issue(rows_a, dst_a, ssem_a)

      @pl.when(h1 < hcount)
      def _b_work():
        drain_gathers(src_b, rows_b, gsem_b)
        scale(rows_b, w_b)
        scatter_issue(rows_b, dst_b, ssem_b)

      scatter_drain(rows_a, dst_a, ssem_a)

      @pl.when(h2 < hcount)
      def _a_next():
        stage_issue(hstart + h2, src_a, dst_a, w_a, stg_a)
        stage_wait(src_a, dst_a, w_a, stg_a)
        fire(src_a, rows_a, gsem_a)

      @pl.when(h1 < hcount)
      def _b_out():
        scatter_drain(rows_b, dst_b, ssem_b)

      @pl.when(h3 < hcount)
      def _b_next():
        stage_issue(hstart + h3, src_b, dst_b, w_b, stg_b)

    plsc.subcore_barrier()

    for ss in range(NS):
      @pl.when(s == ss)
      def _write_range(start=_STARTS[ss], size=_SPLIT[ss]):
        pltpu.sync_copy(agg_sh.at[pl.ds(start, size)],
                        agg_hbm.at[c, pl.ds(start, size)])

  return k(edge4d, w1d, emb)


N8 = N // 8


def _tc_epilogue(agg128, emb128, wr_big, wo_big, b128):
  def body(agg_ref, emb_ref, wr_ref, wo_ref, b_ref, out_ref):
    a = agg_ref[0] + agg_ref[1]
    acc = jnp.dot(a, wr_ref[...], preferred_element_type=jnp.float32)
    acc += jnp.dot(emb_ref[...], wo_ref[...], preferred_element_type=jnp.float32)
    acc += b_ref[...]
    out_ref[...] = jnp.maximum(acc, 0.0)

  return pl.pallas_call(
      body,
      out_shape=jax.ShapeDtypeStruct((N8, 128), jnp.float32),
  )(agg128, emb128, wr_big, wo_big, b128)


@jax.jit
def kernel(edge_index, edge_weight, emb, W_rel, W_root, b):
  edge4d = edge_index.reshape(2, QBLKS, SUPER, SUB)
  agg = _sc_aggregate(edge4d, edge_weight, emb)
  eye8 = jnp.eye(8, dtype=jnp.float32)
  wr_big = jnp.kron(eye8, W_rel)
  wo_big = jnp.kron(eye8, W_root)
  b128 = jnp.tile(b, 8).reshape(1, 128)
  out128 = _tc_epilogue(agg.reshape(NC, N8, 128), emb.reshape(N8, 128),
                        wr_big, wo_big, b128)
  return out128.reshape(N, D)


# speedup vs baseline: 1.1253x; 1.1253x over previous
import functools

import jax
import jax.numpy as jnp
from jax import lax
from jax.experimental import pallas as pl
from jax.experimental.pallas import tpu as pltpu
from jax.experimental.pallas import tpu_sc as plsc

N = 100000
E = 3200000
D = 16

NC = 2
NS = 16
NW = NC * NS

SUB = 128
SUPER = 8
SUPER_C = 4
CHUNK_E = SUPER_C * SUB
QBLKS = E // (SUPER * SUB)
HB = E // CHUNK_E
HBASE = HB // NW
HEXTRA = HB - HBASE * NW

_SPLIT = [6248] * (NS - 1) + [N - 6248 * (NS - 1)]
_STARTS = [6248 * i for i in range(NS)]


def _sc_aggregate(edge4d, w1d, emb):
  mesh = plsc.VectorSubcoreMesh(core_axis_name="c", subcore_axis_name="s")

  @functools.partial(
      pl.kernel,
      out_type=jax.ShapeDtypeStruct((NC, N, D), jnp.float32),
      mesh=mesh,
      scratch_types=[
          pltpu.VMEM_SHARED((N, D), jnp.float32),
          pltpu.VMEM((SUPER_C, SUB), jnp.int32),
          pltpu.VMEM((SUPER_C, SUB), jnp.int32),
          pltpu.VMEM((CHUNK_E,), jnp.float32),
          pltpu.VMEM((CHUNK_E, D), jnp.float32),
          pltpu.VMEM((SUPER_C, SUB), jnp.int32),
          pltpu.VMEM((SUPER_C, SUB), jnp.int32),
          pltpu.VMEM((CHUNK_E,), jnp.float32),
          pltpu.VMEM((CHUNK_E, D), jnp.float32),
          pltpu.SemaphoreType.DMA,
          pltpu.SemaphoreType.DMA,
          pltpu.SemaphoreType.DMA,
          pltpu.SemaphoreType.DMA,
          pltpu.SemaphoreType.DMA,
          pltpu.SemaphoreType.DMA,
      ],
      compiler_params=pltpu.CompilerParams(use_tc_tiling_on_sc=False),
  )
  def k(edge_hbm, w_hbm, emb_hbm, agg_hbm,
        agg_sh, src_a, dst_a, w_a, rows_a, src_b, dst_b, w_b, rows_b,
        gsem_a, gsem_b, ssem_a, ssem_b, stg_a, stg_b):
    c = lax.axis_index("c")
    s = lax.axis_index("s")
    wid = c * NS + s

    @pl.loop(0, CHUNK_E)
    def _zero_buf(i):
      rows_a[i, :] = jnp.zeros((D,), jnp.float32)

    for ss in range(NS):
      @pl.when(s == ss)
      def _zero_range(start=_STARTS[ss], size=_SPLIT[ss]):
        full, rem = size // CHUNK_E, size % CHUNK_E
        for kk in range(full):
          pltpu.sync_copy(rows_a.at[pl.ds(0, CHUNK_E)],
                          agg_sh.at[pl.ds(start + kk * CHUNK_E, CHUNK_E)])
        if rem:
          pltpu.sync_copy(rows_a.at[pl.ds(0, rem)],
                          agg_sh.at[pl.ds(start + full * CHUNK_E, rem)])
    plsc.subcore_barrier()

    hstart = HBASE * wid + jnp.minimum(wid, HEXTRA)
    hcount = HBASE + jnp.where(wid < HEXTRA, 1, 0)
    npairs = (hcount + 1) // 2

    def stage_issue(h, src_v, dst_v, w_v, sem):
      q = h // 2
      half = (h % 2) * SUPER_C
      pltpu.async_copy(edge_hbm.at[0, q, pl.ds(half, SUPER_C)], src_v, sem)
      pltpu.async_copy(edge_hbm.at[1, q, pl.ds(half, SUPER_C)], dst_v, sem)
      pltpu.async_copy(w_hbm.at[pl.ds(h * CHUNK_E, CHUNK_E)], w_v, sem)

    def stage_wait(src_v, dst_v, w_v, sem):
      pltpu.make_async_copy(edge_hbm.at[0, 0, pl.ds(0, SUPER_C)], src_v, sem).wait()
      pltpu.make_async_copy(edge_hbm.at[1, 0, pl.ds(0, SUPER_C)], dst_v, sem).wait()
      pltpu.make_async_copy(w_hbm.at[pl.ds(0, CHUNK_E)], w_v, sem).wait()

    def fire(src_v, rows_v, sem):
      @pl.loop(0, SUPER_C)
      def _f(j):
        pltpu.async_copy(emb_hbm.at[src_v.at[j]],
                         rows_v.at[pl.ds(j * SUB, SUB)], sem)

    def drain_gathers(src_v, rows_v, sem):
      @pl.loop(0, SUPER_C)
      def _d(j):
        pltpu.make_async_copy(emb_hbm.at[src_v.at[j]],
                              rows_v.at[pl.ds(j * SUB, SUB)], sem).wait()

    def process(rows_v, w_v, dst_v, sem):
      for j in range(SUPER_C):
        @pl.loop(0, SUB // 16)
        def _t(t, j=j):
          base = j * SUB + t * 16
          w16 = w_v[pl.ds(base, 16)]
          for e in range(16):
            rows_v[base + e, :] = rows_v[base + e, :] * jnp.broadcast_to(
                w16[e], (D,))
        pltpu.async_copy(rows_v.at[pl.ds(j * SUB, SUB)],
                         agg_sh.at[dst_v.at[j]], sem, add=True)

    def scatter_drain(rows_v, dst_v, sem):
      @pl.loop(0, SUPER_C)
      def _s(j):
        pltpu.make_async_copy(rows_v.at[pl.ds(j * SUB, SUB)],
                              agg_sh.at[dst_v.at[j]], sem).wait()

    stage_issue(hstart, src_a, dst_a, w_a, stg_a)
    stage_wait(src_a, dst_a, w_a, stg_a)
    fire(src_a, rows_a, gsem_a)
    stage_issue(hstart + 1, src_b, dst_b, w_b, stg_b)

    @pl.loop(0, npairs)
    def _pair(t):
      h0 = 2 * t
      h1 = h0 + 1
      h2 = h0 + 2
      h3 = h0 + 3

      drain_gathers(src_a, rows_a, gsem_a)

      @pl.when(h1 < hcount)
      def _b_in():
        stage_wait(src_b, dst_b, w_b, stg_b)
        fire(src_b, rows_b, gsem_b)

      process(rows_a, w_a, dst_a, ssem_a)
      scatter_drain(rows_a, dst_a, ssem_a)

      @pl.when(h2 < hcount)
      def _a_stage():
        stage_issue(hstart + h2, src_a, dst_a, w_a, stg_a)

      @pl.when(h1 < hcount)
      def _b_work():
        drain_gathers(src_b, rows_b, gsem_b)
        process(rows_b, w_b, dst_b, ssem_b)

      @pl.when(h2 < hcount)
      def _a_next():
        stage_wait(src_a, dst_a, w_a, stg_a)
        fire(src_a, rows_a, gsem_a)

      @pl.when(h1 < hcount)
      def _b_out():
        scatter_drain(rows_b, dst_b, ssem_b)

      @pl.when(h3 < hcount)
      def _b_next():
        stage_issue(hstart + h3, src_b, dst_b, w_b, stg_b)

    plsc.subcore_barrier()

    for ss in range(NS):
      @pl.when(s == ss)
      def _write_range(start=_STARTS[ss], size=_SPLIT[ss]):
        pltpu.sync_copy(agg_sh.at[pl.ds(start, size)],
                        agg_hbm.at[c, pl.ds(start, size)])

  return k(edge4d, w1d, emb)


N8 = N // 8


def _tc_epilogue(agg128, emb128, wr_big, wo_big, b128):
  def body(agg_ref, emb_ref, wr_ref, wo_ref, b_ref, out_ref):
    a = agg_ref[0] + agg_ref[1]
    acc = jnp.dot(a, wr_ref[...], preferred_element_type=jnp.float32)
    acc += jnp.dot(emb_ref[...], wo_ref[...], preferred_element_type=jnp.float32)
    acc += b_ref[...]
    out_ref[...] = jnp.maximum(acc, 0.0)

  return pl.pallas_call(
      body,
      out_shape=jax.ShapeDtypeStruct((N8, 128), jnp.float32),
  )(agg128, emb128, wr_big, wo_big, b128)


@jax.jit
def kernel(edge_index, edge_weight, emb, W_rel, W_root, b):
  edge4d = edge_index.reshape(2, QBLKS, SUPER, SUB)
  agg = _sc_aggregate(edge4d, edge_weight, emb)
  eye8 = jnp.eye(8, dtype=jnp.float32)
  wr_big = jnp.kron(eye8, W_rel)
  wo_big = jnp.kron(eye8, W_root)
  b128 = jnp.tile(b, 8).reshape(1, 128)
  out128 = _tc_epilogue(agg.reshape(NC, N8, 128), emb.reshape(N8, 128),
                        wr_big, wo_big, b128)
  return out128.reshape(N, D)


# speedup vs baseline: 1.1544x; 1.0259x over previous
import functools

import jax
import jax.numpy as jnp
from jax import lax
from jax.experimental import pallas as pl
from jax.experimental.pallas import tpu as pltpu
from jax.experimental.pallas import tpu_sc as plsc

N = 100000
E = 3200000
D = 16

NC = 2
NS = 16
NW = NC * NS

SUB = 128
SUPER = 8
SUPER_C = 4
CHUNK_E = SUPER_C * SUB
QBLKS = E // (SUPER * SUB)
HB = E // CHUNK_E
HBASE = HB // NW
HEXTRA = HB - HBASE * NW

_SPLIT = [6248] * (NS - 1) + [N - 6248 * (NS - 1)]
_STARTS = [6248 * i for i in range(NS)]


def _sc_aggregate(edge4d, w1d, emb):
  mesh = plsc.VectorSubcoreMesh(core_axis_name="c", subcore_axis_name="s")

  @functools.partial(
      pl.kernel,
      out_type=jax.ShapeDtypeStruct((NC, N, D), jnp.float32),
      mesh=mesh,
      scratch_types=[
          pltpu.VMEM_SHARED((N, D), jnp.float32),
          pltpu.VMEM((SUPER_C, SUB), jnp.int32),
          pltpu.VMEM((SUPER_C, SUB), jnp.int32),
          pltpu.VMEM((CHUNK_E,), jnp.float32),
          pltpu.VMEM((CHUNK_E, D), jnp.float32),
          pltpu.VMEM((SUPER_C, SUB), jnp.int32),
          pltpu.VMEM((SUPER_C, SUB), jnp.int32),
          pltpu.VMEM((CHUNK_E,), jnp.float32),
          pltpu.VMEM((CHUNK_E, D), jnp.float32),
          pltpu.VMEM((SUPER_C, SUB), jnp.int32),
          pltpu.VMEM((SUPER_C, SUB), jnp.int32),
          pltpu.VMEM((CHUNK_E,), jnp.float32),
          pltpu.VMEM((CHUNK_E, D), jnp.float32),
          pltpu.SemaphoreType.DMA,
          pltpu.SemaphoreType.DMA,
          pltpu.SemaphoreType.DMA,
          pltpu.SemaphoreType.DMA,
          pltpu.SemaphoreType.DMA,
          pltpu.SemaphoreType.DMA,
          pltpu.SemaphoreType.DMA,
          pltpu.SemaphoreType.DMA,
          pltpu.SemaphoreType.DMA,
      ],
      compiler_params=pltpu.CompilerParams(use_tc_tiling_on_sc=False),
  )
  def k(edge_hbm, w_hbm, emb_hbm, agg_hbm,
        agg_sh, src_a, dst_a, w_a, rows_a, src_b, dst_b, w_b, rows_b,
        src_c, dst_c, w_c, rows_c,
        gsem_a, gsem_b, gsem_c, ssem_a, ssem_b, ssem_c,
        stg_a, stg_b, stg_c):
    c = lax.axis_index("c")
    s = lax.axis_index("s")
    wid = c * NS + s

    @pl.loop(0, CHUNK_E)
    def _zero_buf(i):
      rows_a[i, :] = jnp.zeros((D,), jnp.float32)

    for ss in range(NS):
      @pl.when(s == ss)
      def _zero_range(start=_STARTS[ss], size=_SPLIT[ss]):
        full, rem = size // CHUNK_E, size % CHUNK_E
        for kk in range(full):
          pltpu.sync_copy(rows_a.at[pl.ds(0, CHUNK_E)],
                          agg_sh.at[pl.ds(start + kk * CHUNK_E, CHUNK_E)])
        if rem:
          pltpu.sync_copy(rows_a.at[pl.ds(0, rem)],
                          agg_sh.at[pl.ds(start + full * CHUNK_E, rem)])
    plsc.subcore_barrier()

    hstart = HBASE * wid + jnp.minimum(wid, HEXTRA)
    hcount = HBASE + jnp.where(wid < HEXTRA, 1, 0)

    def stage_issue(h, src_v, dst_v, w_v, sem):
      q = h // 2
      half = (h % 2) * SUPER_C
      pltpu.async_copy(edge_hbm.at[0, q, pl.ds(half, SUPER_C)], src_v, sem)
      pltpu.async_copy(edge_hbm.at[1, q, pl.ds(half, SUPER_C)], dst_v, sem)
      pltpu.async_copy(w_hbm.at[pl.ds(h * CHUNK_E, CHUNK_E)], w_v, sem)

    def stage_wait(src_v, dst_v, w_v, sem):
      pltpu.make_async_copy(edge_hbm.at[0, 0, pl.ds(0, SUPER_C)], src_v, sem).wait()
      pltpu.make_async_copy(edge_hbm.at[1, 0, pl.ds(0, SUPER_C)], dst_v, sem).wait()
      pltpu.make_async_copy(w_hbm.at[pl.ds(0, CHUNK_E)], w_v, sem).wait()

    def fire(src_v, rows_v, sem):
      @pl.loop(0, SUPER_C)
      def _f(j):
        pltpu.async_copy(emb_hbm.at[src_v.at[j]],
                         rows_v.at[pl.ds(j * SUB, SUB)], sem)

    def drain_gathers(src_v, rows_v, sem):
      @pl.loop(0, SUPER_C)
      def _d(j):
        pltpu.make_async_copy(emb_hbm.at[src_v.at[j]],
                              rows_v.at[pl.ds(j * SUB, SUB)], sem).wait()

    def process(rows_v, w_v, dst_v, sem):
      for j in range(SUPER_C):
        @pl.loop(0, SUB // 16)
        def _t(t, j=j):
          base = j * SUB + t * 16
          w16 = w_v[pl.ds(base, 16)]
          for e in range(16):
            rows_v[base + e, :] = rows_v[base + e, :] * jnp.broadcast_to(
                w16[e], (D,))
        pltpu.async_copy(rows_v.at[pl.ds(j * SUB, SUB)],
                         agg_sh.at[dst_v.at[j]], sem, add=True)

    def scatter_drain(rows_v, dst_v, sem):
      @pl.loop(0, SUPER_C)
      def _s(j):
        pltpu.make_async_copy(rows_v.at[pl.ds(j * SUB, SUB)],
                              agg_sh.at[dst_v.at[j]], sem).wait()

    stage_issue(hstart, src_a, dst_a, w_a, stg_a)
    stage_wait(src_a, dst_a, w_a, stg_a)
    fire(src_a, rows_a, gsem_a)
    stage_issue(hstart + 1, src_b, dst_b, w_b, stg_b)
    stage_issue(hstart + 2, src_c, dst_c, w_c, stg_c)

    ntriples = 65

    @pl.loop(0, ntriples)
    def _triple(t):
      h3 = 3 * t + 3
      h4 = 3 * t + 4
      h5 = 3 * t + 5

      drain_gathers(src_a, rows_a, gsem_a)
      stage_wait(src_b, dst_b, w_b, stg_b)
      fire(src_b, rows_b, gsem_b)
      process(rows_a, w_a, dst_a, ssem_a)

      drain_gathers(src_b, rows_b, gsem_b)
      stage_wait(src_c, dst_c, w_c, stg_c)
      fire(src_c, rows_c, gsem_c)
      process(rows_b, w_b, dst_b, ssem_b)

      scatter_drain(rows_a, dst_a, ssem_a)

      @pl.when(h3 < hcount)
      def _a_stage():
        stage_issue(hstart + h3, src_a, dst_a, w_a, stg_a)

      drain_gathers(src_c, rows_c, gsem_c)
      process(rows_c, w_c, dst_c, ssem_c)

      scatter_drain(rows_b, dst_b, ssem_b)

      @pl.when(h4 < hcount)
      def _b_stage():
        stage_issue(hstart + h4, src_b, dst_b, w_b, stg_b)

      @pl.when(h3 < hcount)
      def _a_fire():
        stage_wait(src_a, dst_a, w_a, stg_a)
        fire(src_a, rows_a, gsem_a)

      scatter_drain(rows_c, dst_c, ssem_c)

      @pl.when(h5 < hcount)
      def _c_stage():
        stage_issue(hstart + h5, src_c, dst_c, w_c, stg_c)

    @pl.when(hcount > 3 * ntriples)
    def _tail():
      drain_gathers(src_a, rows_a, gsem_a)
      process(rows_a, w_a, dst_a, ssem_a)
      scatter_drain(rows_a, dst_a, ssem_a)

    plsc.subcore_barrier()

    for ss in range(NS):
      @pl.when(s == ss)
      def _write_range(start=_STARTS[ss], size=_SPLIT[ss]):
        pltpu.sync_copy(agg_sh.at[pl.ds(start, size)],
                        agg_hbm.at[c, pl.ds(start, size)])

  return k(edge4d, w1d, emb)


N8 = N // 8


def _tc_epilogue(agg128, emb128, wr_big, wo_big, b128):
  def body(agg_ref, emb_ref, wr_ref, wo_ref, b_ref, out_ref):
    a = agg_ref[0] + agg_ref[1]
    acc = jnp.dot(a, wr_ref[...], preferred_element_type=jnp.float32)
    acc += jnp.dot(emb_ref[...], wo_ref[...], preferred_element_type=jnp.float32)
    acc += b_ref[...]
    out_ref[...] = jnp.maximum(acc, 0.0)

  return pl.pallas_call(
      body,
      out_shape=jax.ShapeDtypeStruct((N8, 128), jnp.float32),
  )(agg128, emb128, wr_big, wo_big, b128)


@jax.jit
def kernel(edge_index, edge_weight, emb, W_rel, W_root, b):
  edge4d = edge_index.reshape(2, QBLKS, SUPER, SUB)
  agg = _sc_aggregate(edge4d, edge_weight, emb)
  eye8 = jnp.eye(8, dtype=jnp.float32)
  wr_big = jnp.kron(eye8, W_rel)
  wo_big = jnp.kron(eye8, W_root)
  b128 = jnp.tile(b, 8).reshape(1, 128)
  out128 = _tc_epilogue(agg.reshape(NC, N8, 128), emb.reshape(N8, 128),
                        wr_big, wo_big, b128)
  return out128.reshape(N, D)
